# baseline (speedup 1.0000x reference)
import functools

import jax
import jax.numpy as jnp
from jax import lax
from jax.experimental import pallas as pl
from jax.experimental.pallas import tpu as pltpu
from jax.experimental.pallas import tpu_sc as plsc

N_NODES = 10000
N_EDGES = 320000
DIM = 128
HD = DIM // 2

NC = 2
NS = 16
CH = 128
NCHUNK = 157
EPT = NCHUNK * CH
E_PAD = NS * EPT
NBUF = 6
N_PAD = 10240
RPT = N_PAD // NS
ZCH = 120
LANES = 16
GARBAGE = N_PAD - 1


def _sc_aggregate(x2, e4):
    mesh = plsc.VectorSubcoreMesh(core_axis_name="c", subcore_axis_name="s")

    @functools.partial(
        pl.kernel,
        mesh=mesh,
        out_type=jax.ShapeDtypeStruct((N_PAD, DIM), jnp.float32),
        compiler_params=pltpu.CompilerParams(use_tc_tiling_on_sc=False),
        scratch_types=[
            pltpu.VMEM((NCHUNK, CH), jnp.int32),
            pltpu.VMEM((NCHUNK, CH), jnp.int32),
            pltpu.VMEM((NBUF, CH, HD), jnp.float32),
            pltpu.VMEM_SHARED((N_PAD, HD), jnp.float32),
            pltpu.SemaphoreType.DMA,
            pltpu.SemaphoreType.DMA,
        ],
    )
    def k(x_hbm, e_hbm, out_hbm, sidx, didx, rows, acc, gsem, ssem):
        c = lax.axis_index("c")
        s = lax.axis_index("s")

        pltpu.sync_copy(e_hbm.at[0, s], sidx)
        pltpu.sync_copy(e_hbm.at[1, s], didx)

        def trow(i):
            def tcol(j, _):
                sl = pl.ds(j * LANES, LANES)
                sidx[i, sl] = 2 * sidx[i, sl] + c
                return 0
            lax.fori_loop(0, CH // LANES, tcol, 0)

        for p in range(NBUF - 1):
            trow(p)
            pltpu.async_copy(x_hbm.at[sidx.at[p]], rows.at[p], gsem)

        def zbody(t, _):
            i = t // (HD // LANES)
            j = t % (HD // LANES)
            rows[NBUF - 1, i, pl.ds(j * LANES, LANES)] = jnp.zeros(
                (LANES,), jnp.float32)
            return 0
        lax.fori_loop(0, ZCH * (HD // LANES), zbody, 0)
        for j in range(RPT // ZCH + 1):
            rr = min(ZCH, RPT - j * ZCH)
            pltpu.sync_copy(rows.at[NBUF - 1, pl.ds(0, rr)],
                            acc.at[pl.ds(s * RPT + j * ZCH, rr)])
        plsc.subcore_barrier()

        def chunk_body(i, _):
            b = lax.rem(i, NBUF)
            pltpu.make_async_copy(
                x_hbm.at[sidx.at[i]], rows.at[b], gsem).wait()
            pltpu.async_copy(rows.at[b], acc.at[didx.at[i]], ssem, add=True)

            nxt = i + NBUF - 1
            nb = lax.rem(nxt, NBUF)

            @pl.when(nxt < NCHUNK)
            def _prefetch():
                @pl.when(i >= 1)
                def _drain_one():
                    pltpu.make_async_copy(
                        rows.at[nb], acc.at[didx.at[i]], ssem).wait()
                trow(nxt)
                pltpu.async_copy(x_hbm.at[sidx.at[nxt]], rows.at[nb], gsem)
            return 0
        lax.fori_loop(0, NCHUNK, chunk_body, 0)

        for p in range(NBUF):
            pltpu.make_async_copy(
                rows.at[p], acc.at[didx.at[0]], ssem).wait()

        plsc.subcore_barrier()
        pltpu.sync_copy(acc.at[pl.ds(s * RPT, RPT)],
                        out_hbm.at[pl.ds(s * RPT, RPT), pl.ds(c * HD, HD)])

    return k(x2, e4)


def _tc_combine(x, agg, W1t, W2t):
    BR = 10000
    grid = N_NODES // BR

    def body(x_ref, a_ref, w1_ref, w2_ref, o_ref):
        acc = jnp.dot(x_ref[...], w1_ref[...],
                      preferred_element_type=jnp.float32)
        acc += jnp.dot(a_ref[...], w2_ref[...],
                       preferred_element_type=jnp.float32)
        o_ref[...] = jnp.maximum(acc, 0.0)

    return pl.pallas_call(
        body,
        grid=(grid,),
        in_specs=[
            pl.BlockSpec((BR, DIM), lambda i: (i, 0)),
            pl.BlockSpec((BR, DIM), lambda i: (i, 0)),
            pl.BlockSpec((DIM, DIM), lambda i: (0, 0)),
            pl.BlockSpec((DIM, DIM), lambda i: (0, 0)),
        ],
        out_specs=pl.BlockSpec((BR, DIM), lambda i: (i, 0)),
        out_shape=jax.ShapeDtypeStruct((N_NODES, DIM), jnp.float32),
    )(x, agg, W1t, W2t)


def kernel(x, edge_index, W1, W2):
    edges = edge_index.astype(jnp.int32)
    npad = E_PAD - N_EDGES
    pad_dst = N_NODES + jax.lax.rem(jnp.arange(npad, dtype=jnp.int32),
                                    jnp.int32(N_PAD - N_NODES))
    pad_src = jax.lax.rem(jnp.arange(npad, dtype=jnp.int32),
                          jnp.int32(N_NODES))
    pad = jnp.stack([pad_src, pad_dst])
    e4 = jnp.concatenate([edges, pad], axis=1).reshape(2, NS, NCHUNK, CH)
    x2 = x.reshape(2 * N_NODES, HD)
    agg = _sc_aggregate(x2, e4)
    return _tc_combine(x, agg, W1.T, W2.T)


# speedup vs baseline: 1.0045x; 1.0045x over previous
import functools

import jax
import jax.numpy as jnp
from jax import lax
from jax.experimental import pallas as pl
from jax.experimental.pallas import tpu as pltpu
from jax.experimental.pallas import tpu_sc as plsc

N_NODES = 10000
N_EDGES = 320000
DIM = 128
HD = DIM // 2

NC = 2
NS = 16
CH = 128
NCHUNK = 157
EPT = NCHUNK * CH
E_PAD = NS * EPT
NBUF = 6
N_PAD = 10240
RPT = N_PAD // NS
ZCH = 120
LANES = 16
GARBAGE = N_PAD - 1


def _sc_aggregate(x2, e4):
    mesh = plsc.VectorSubcoreMesh(core_axis_name="c", subcore_axis_name="s")

    @functools.partial(
        pl.kernel,
        mesh=mesh,
        out_type=jax.ShapeDtypeStruct((N_PAD, DIM), jnp.float32),
        compiler_params=pltpu.CompilerParams(use_tc_tiling_on_sc=False),
        scratch_types=[
            pltpu.VMEM((NCHUNK, CH), jnp.int32),
            pltpu.VMEM((NCHUNK, CH), jnp.int32),
            pltpu.VMEM((NBUF, CH, HD), jnp.float32),
            pltpu.VMEM_SHARED((N_PAD, HD), jnp.float32),
            pltpu.SemaphoreType.DMA,
            pltpu.SemaphoreType.DMA,
        ],
    )
    def k(x_hbm, e_hbm, out_hbm, sidx, didx, rows, acc, gsem, ssem):
        c = lax.axis_index("c")
        s = lax.axis_index("s")

        pltpu.sync_copy(e_hbm.at[0, s], sidx)
        pltpu.sync_copy(e_hbm.at[1, s], didx)

        def trow(i):
            def tcol(j, _):
                sl = pl.ds(j * LANES, LANES)
                sidx[i, sl] = 2 * sidx[i, sl] + c
                return 0
            lax.fori_loop(0, CH // LANES, tcol, 0)

        for p in range(NBUF - 1):
            trow(p)
            pltpu.async_copy(x_hbm.at[sidx.at[p]], rows.at[p], gsem)

        def zbody(t, _):
            i = t // (HD // LANES)
            j = t % (HD // LANES)
            rows[NBUF - 1, i, pl.ds(j * LANES, LANES)] = jnp.zeros(
                (LANES,), jnp.float32)
            return 0
        lax.fori_loop(0, ZCH * (HD // LANES), zbody, 0)
        for j in range(RPT // ZCH + 1):
            rr = min(ZCH, RPT - j * ZCH)
            pltpu.sync_copy(rows.at[NBUF - 1, pl.ds(0, rr)],
                            acc.at[pl.ds(s * RPT + j * ZCH, rr)])
        plsc.subcore_barrier()

        def chunk_body(i, _):
            b = lax.rem(i, NBUF)
            pltpu.make_async_copy(
                x_hbm.at[sidx.at[i]], rows.at[b], gsem).wait()
            pltpu.async_copy(rows.at[b], acc.at[didx.at[i]], ssem, add=True)

            nxt = i + NBUF - 1
            nb = lax.rem(nxt, NBUF)

            @pl.when(nxt < NCHUNK)
            def _prefetch():
                @pl.when(i >= 1)
                def _drain_one():
                    pltpu.make_async_copy(
                        rows.at[nb], acc.at[didx.at[i]], ssem).wait()
                trow(nxt)
                pltpu.async_copy(x_hbm.at[sidx.at[nxt]], rows.at[nb], gsem)
            return 0
        lax.fori_loop(0, NCHUNK, chunk_body, 0)

        for p in range(NBUF):
            pltpu.make_async_copy(
                rows.at[p], acc.at[didx.at[0]], ssem).wait()

        plsc.subcore_barrier()
        pltpu.sync_copy(acc.at[pl.ds(s * RPT, RPT)],
                        out_hbm.at[pl.ds(s * RPT, RPT), pl.ds(c * HD, HD)])

    return k(x2, e4)


def _tc_combine(x, agg, W1t, W2t):
    BR = 5000
    grid = N_NODES // BR

    def body(x_ref, a_ref, w1_ref, w2_ref, o_ref):
        acc = jnp.dot(x_ref[...], w1_ref[...],
                      preferred_element_type=jnp.float32)
        acc += jnp.dot(a_ref[...], w2_ref[...],
                       preferred_element_type=jnp.float32)
        o_ref[...] = jnp.maximum(acc, 0.0)

    return pl.pallas_call(
        body,
        grid=(grid,),
        in_specs=[
            pl.BlockSpec((BR, DIM), lambda i: (i, 0)),
            pl.BlockSpec((BR, DIM), lambda i: (i, 0)),
            pl.BlockSpec((DIM, DIM), lambda i: (0, 0)),
            pl.BlockSpec((DIM, DIM), lambda i: (0, 0)),
        ],
        out_specs=pl.BlockSpec((BR, DIM), lambda i: (i, 0)),
        out_shape=jax.ShapeDtypeStruct((N_NODES, DIM), jnp.float32),
    )(x, agg, W1t, W2t)


def kernel(x, edge_index, W1, W2):
    edges = edge_index.astype(jnp.int32)
    npad = E_PAD - N_EDGES
    pad_dst = N_NODES + jax.lax.rem(jnp.arange(npad, dtype=jnp.int32),
                                    jnp.int32(N_PAD - N_NODES))
    pad_src = jax.lax.rem(jnp.arange(npad, dtype=jnp.int32),
                          jnp.int32(N_NODES))
    pad = jnp.stack([pad_src, pad_dst])
    e4 = jnp.concatenate([edges, pad], axis=1).reshape(2, NS, NCHUNK, CH)
    x2 = x.reshape(2 * N_NODES, HD)
    agg = _sc_aggregate(x2, e4)
    return _tc_combine(x, agg, W1.T, W2.T)
